# baseline (speedup 1.0000x reference)
import functools
import jax
import jax.numpy as jnp
from jax import lax
from jax.experimental import pallas as pl
from jax.experimental.pallas import tpu as pltpu, tpu_sc as plsc


def kernel(x, table):
    batch, seq_len = x.shape
    _, dim = table.shape
    width = seq_len * dim
    flat = table[:seq_len].reshape(width)

    info = plsc.get_sparse_core_info()
    nc, ns = info.num_cores, info.num_subcores
    nw = nc * ns
    rows_per_w = batch // nw
    rep = 8
    n_chunks = rows_per_w // rep

    mesh = plsc.VectorSubcoreMesh(core_axis_name="c", subcore_axis_name="s")

    @functools.partial(
        pl.kernel,
        mesh=mesh,
        out_type=jax.ShapeDtypeStruct((batch, width), jnp.float32),
        scratch_types=[
            pltpu.VMEM((rep, width), jnp.float32),
            pltpu.SemaphoreType.DMA,
        ],
    )
    def bcast(table_hbm, out_hbm, buf, sem):
        wid = lax.axis_index("s") * nc + lax.axis_index("c")
        base = wid * rows_per_w
        for r in range(rep):
            pltpu.sync_copy(table_hbm, buf.at[r])
        copies = [
            pltpu.async_copy(buf, out_hbm.at[pl.ds(base + k * rep, rep)], sem)
            for k in range(n_chunks)
        ]
        for c in copies:
            c.wait()

    out = bcast(flat)
    return out.reshape(batch, seq_len, dim)


# speedup vs baseline: 1.1836x; 1.1836x over previous
import jax
import jax.numpy as jnp
from jax.experimental import pallas as pl
from jax.experimental.pallas import tpu as pltpu

_BB = 128
_K = 8


def _body(t_ref, o_ref, scratch, sems):
    scratch[...] = jnp.broadcast_to(t_ref[...], scratch.shape)
    nblk = o_ref.shape[0] // _BB
    copies = [
        pltpu.make_async_copy(scratch, o_ref.at[pl.ds(i * _BB, _BB)], sems.at[i % _K])
        for i in range(nblk)
    ]
    for c in copies:
        c.start()
    for c in copies:
        c.wait()


def kernel(x, table):
    batch, seq_len = x.shape
    _, dim = table.shape
    width = seq_len * dim
    flat = table[:seq_len].reshape(1, width)

    out = pl.pallas_call(
        _body,
        in_specs=[pl.BlockSpec(memory_space=pltpu.VMEM)],
        out_specs=pl.BlockSpec(memory_space=pl.ANY),
        out_shape=jax.ShapeDtypeStruct((batch, width), jnp.float32),
        scratch_shapes=[
            pltpu.VMEM((_BB, width), jnp.float32),
            pltpu.SemaphoreType.DMA((_K,)),
        ],
    )(flat)
    return out.reshape(batch, seq_len, dim)
